# baseline (speedup 1.0000x reference)
import jax
import jax.numpy as jnp
from jax.experimental import pallas as pl
from jax.experimental.pallas import tpu as pltpu

D_MODEL = 1024
BS = 512


def _posenc_kernel(t_ref, x_ref, pe_ref, te_ref, o_ref):
    j = pl.program_id(1)
    pe_blk = pe_ref[0, pl.ds(j * BS, BS), :]
    o_ref[...] = x_ref[...] + pe_blk[None] + te_ref[...]


def kernel(x, t, pe, te):
    B, S, D = x.shape
    te2 = te.reshape(te.shape[1], 1, D)
    grid = (B, S // BS)
    out = pl.pallas_call(
        _posenc_kernel,
        grid_spec=pltpu.PrefetchScalarGridSpec(
            num_scalar_prefetch=1,
            grid=grid,
            in_specs=[
                pl.BlockSpec((1, BS, D), lambda b, j, t_ref: (b, j, 0)),
                pl.BlockSpec((1, S, D), lambda b, j, t_ref: (0, 0, 0)),
                pl.BlockSpec((1, 1, D), lambda b, j, t_ref: (t_ref[b], 0, 0)),
            ],
            out_specs=pl.BlockSpec((1, BS, D), lambda b, j, t_ref: (b, j, 0)),
        ),
        out_shape=jax.ShapeDtypeStruct((B, S, D), x.dtype),
    )(t, x, pe, te2)
    return out


# speedup vs baseline: 1.4493x; 1.4493x over previous
import jax
import jax.numpy as jnp
from jax.experimental import pallas as pl
from jax.experimental.pallas import tpu as pltpu

D = 1024
CH = 512
NBUF = 4


def _posenc_kernel(t_ref, x_hbm, pe_hbm, te_hbm, o_hbm,
                   xbuf, obuf, pebuf, tebuf,
                   xsem, osem, pesem, tesem):
    B, S, _ = x_hbm.shape
    ncpb = S // CH
    nch = B * ncpb

    def x_cp(i):
        b, c = i // ncpb, i % ncpb
        return pltpu.make_async_copy(
            x_hbm.at[b, pl.ds(c * CH, CH), :], xbuf.at[i % NBUF],
            xsem.at[i % NBUF])

    def o_cp(i):
        b, c = i // ncpb, i % ncpb
        return pltpu.make_async_copy(
            obuf.at[i % NBUF], o_hbm.at[b, pl.ds(c * CH, CH), :],
            osem.at[i % NBUF])

    def pe_cp(c):
        return pltpu.make_async_copy(
            pe_hbm.at[0, pl.ds(c * CH, CH), :], pebuf.at[c], pesem.at[c])

    def te_cp(b):
        return pltpu.make_async_copy(
            te_hbm.at[0, pl.ds(t_ref[b], 1), :], tebuf.at[pl.ds(b, 1), :],
            tesem)

    for b in range(B):
        te_cp(b).start()
    for i in range(NBUF):
        x_cp(i).start()
    for c in range(ncpb):
        pe_cp(c).start()

    for i in range(nch):
        b, c = i // ncpb, i % ncpb
        if i < ncpb:
            pe_cp(c).wait()
        if i == 0:
            for bb in range(B):
                te_cp(bb).wait()
        if i >= NBUF:
            o_cp(i - NBUF).wait()
        x_cp(i).wait()
        obuf[i % NBUF] = xbuf[i % NBUF] + pebuf[c] + tebuf[pl.ds(b, 1), :]
        o_cp(i).start()
        if i + NBUF < nch:
            x_cp(i + NBUF).start()

    for i in range(nch - NBUF, nch):
        o_cp(i).wait()


def kernel(x, t, pe, te):
    B, S, _ = x.shape
    ncpb = S // CH
    out = pl.pallas_call(
        _posenc_kernel,
        in_specs=[
            pl.BlockSpec(memory_space=pltpu.SMEM),
            pl.BlockSpec(memory_space=pltpu.HBM),
            pl.BlockSpec(memory_space=pltpu.HBM),
            pl.BlockSpec(memory_space=pltpu.HBM),
        ],
        out_specs=pl.BlockSpec(memory_space=pltpu.HBM),
        out_shape=jax.ShapeDtypeStruct((B, S, D), x.dtype),
        scratch_shapes=[
            pltpu.VMEM((NBUF, CH, D), jnp.float32),
            pltpu.VMEM((NBUF, CH, D), jnp.float32),
            pltpu.VMEM((ncpb, CH, D), jnp.float32),
            pltpu.VMEM((8, D), jnp.float32),
            pltpu.SemaphoreType.DMA((NBUF,)),
            pltpu.SemaphoreType.DMA((NBUF,)),
            pltpu.SemaphoreType.DMA((ncpb,)),
            pltpu.SemaphoreType.DMA,
        ],
    )(t, x, pe, te)
    return out


# speedup vs baseline: 1.4555x; 1.0043x over previous
import jax
import jax.numpy as jnp
from jax.experimental import pallas as pl
from jax.experimental.pallas import tpu as pltpu

D = 1024
CH = 256
NBUF = 8


def _posenc_kernel(t_ref, x_hbm, pe_hbm, te_hbm, o_hbm,
                   xbuf, obuf, pebuf, tebuf,
                   xsem, osem, pesem, tesem):
    B, S, _ = x_hbm.shape
    ncpb = S // CH
    nch = B * ncpb

    def x_cp(i):
        b, c = i // ncpb, i % ncpb
        return pltpu.make_async_copy(
            x_hbm.at[b, pl.ds(c * CH, CH), :], xbuf.at[i % NBUF],
            xsem.at[i % NBUF])

    def o_cp(i):
        b, c = i // ncpb, i % ncpb
        return pltpu.make_async_copy(
            obuf.at[i % NBUF], o_hbm.at[b, pl.ds(c * CH, CH), :],
            osem.at[i % NBUF])

    def pe_cp(c):
        return pltpu.make_async_copy(
            pe_hbm.at[0, pl.ds(c * CH, CH), :], pebuf.at[c], pesem.at[c])

    def te_cp(b):
        return pltpu.make_async_copy(
            te_hbm.at[0, pl.ds(t_ref[b], 1), :], tebuf.at[pl.ds(b, 1), :],
            tesem)

    for b in range(B):
        te_cp(b).start()
    for i in range(NBUF):
        x_cp(i).start()
    for c in range(ncpb):
        pe_cp(c).start()

    for i in range(nch):
        b, c = i // ncpb, i % ncpb
        if i < ncpb:
            pe_cp(c).wait()
        if i == 0:
            for bb in range(B):
                te_cp(bb).wait()
        if i >= NBUF:
            o_cp(i - NBUF).wait()
        x_cp(i).wait()
        obuf[i % NBUF] = xbuf[i % NBUF] + pebuf[c] + tebuf[pl.ds(b, 1), :]
        o_cp(i).start()
        if i + NBUF < nch:
            x_cp(i + NBUF).start()

    for i in range(nch - NBUF, nch):
        o_cp(i).wait()


def kernel(x, t, pe, te):
    B, S, _ = x.shape
    ncpb = S // CH
    out = pl.pallas_call(
        _posenc_kernel,
        in_specs=[
            pl.BlockSpec(memory_space=pltpu.SMEM),
            pl.BlockSpec(memory_space=pltpu.HBM),
            pl.BlockSpec(memory_space=pltpu.HBM),
            pl.BlockSpec(memory_space=pltpu.HBM),
        ],
        out_specs=pl.BlockSpec(memory_space=pltpu.HBM),
        out_shape=jax.ShapeDtypeStruct((B, S, D), x.dtype),
        scratch_shapes=[
            pltpu.VMEM((NBUF, CH, D), jnp.float32),
            pltpu.VMEM((NBUF, CH, D), jnp.float32),
            pltpu.VMEM((ncpb, CH, D), jnp.float32),
            pltpu.VMEM((8, D), jnp.float32),
            pltpu.SemaphoreType.DMA((NBUF,)),
            pltpu.SemaphoreType.DMA((NBUF,)),
            pltpu.SemaphoreType.DMA((ncpb,)),
            pltpu.SemaphoreType.DMA,
        ],
    )(t, x, pe, te)
    return out
